# speedup vs baseline: 1.0000x
import jax
import jax.numpy as jnp
from jax.experimental import pallas as pl


def kernel(x, conv1_w, conv1_shift, conv2_w, conv2_shift, conv3_1_w, conv3_1_shift, conv3_2_w, conv3_2_shift, conv4_1_w, conv4_1_shift, conv4_2_w, conv4_2_shift, inc1_fused_w, inc1_fused_shift, inc1_branch1x1_2_w, inc1_branch1x1_2_shift, inc1_branch3x3_w, inc1_branch3x3_shift, inc1_branch3x3_2_w, inc1_branch3x3_2_shift, inc1_branch3x3_3_w, inc1_branch3x3_3_shift, inc2_fused_w, inc2_fused_shift, inc2_branch1x1_2_w, inc2_branch1x1_2_shift, inc2_branch3x3_w, inc2_branch3x3_shift, inc2_branch3x3_2_w, inc2_branch3x3_2_shift, inc2_branch3x3_3_w, inc2_branch3x3_3_shift, inc3_fused_w, inc3_fused_shift, inc3_branch1x1_2_w, inc3_branch1x1_2_shift, inc3_branch3x3_w, inc3_branch3x3_shift, inc3_branch3x3_2_w, inc3_branch3x3_2_shift, inc3_branch3x3_3_w, inc3_branch3x3_3_shift, head0_w, head0_shift, head1_w, head1_shift, head2_w, head2_shift):
    raise NotImplementedError("write your pallas kernel here")


# speedup vs baseline: 40.2243x; 40.2243x over previous
import functools

import numpy as np

import jax
import jax.numpy as jnp
from jax import lax
from jax.experimental import pallas as pl
from jax.experimental.pallas import tpu as pltpu

_BF = jnp.bfloat16
_F32 = jnp.float32


def _gather_rows(w, idx):
    valid = (idx >= 0)
    g = jnp.take(w, jnp.asarray(np.where(valid, idx, 0)), axis=0)
    return jnp.where(jnp.asarray(valid)[:, None], g, jnp.zeros_like(g))


def _conv1_taps():
    idx = np.full((256,), -1, np.int32)
    for ky in range(7):
        for kx in range(7):
            dy, sy = (ky + 1) // 4, (ky + 1) % 4
            dx, sx = (kx + 1) // 4, (kx + 1) % 4
            for c in range(3):
                idx[(dy * 2 + dx) * 64 + sy * 12 + sx * 3 + c] = (ky * 7 + kx) * 3 + c
    return idx


_CONV1_IDX = _conv1_taps()


def _conv2_taps():
    idx = np.full((1728,), -1, np.int32)
    for ky in range(5):
        for kx in range(5):
            dy, sy = ky // 2, ky % 2
            dx, sx = kx // 2, kx % 2
            for c in range(48):
                idx[(dy * 3 + dx) * 192 + sy * 96 + sx * 48 + c] = (ky * 5 + kx) * 48 + c
    return idx


_CONV2_IDX = _conv2_taps()


def _crelu_pack(w, shift):
    w2 = jnp.concatenate([w, -w], axis=1).astype(_BF)
    s2 = jnp.concatenate([shift, -shift], axis=1)
    return w2, s2


def _band(w, s, src_cols, dst_lo, total=128):
    wb = jnp.pad(w[:, :src_cols].astype(_F32),
                 ((0, 0), (dst_lo, total - dst_lo - src_cols)))
    sb = jnp.pad(s[:, :src_cols], ((0, 0), (dst_lo, total - dst_lo - src_cols)))
    return wb.astype(_BF), sb


def _embed_rows(w, src_cols, cin_lo, cin_n, cin_total, dst_lo, total=128):
    a = w[:, :src_cols].astype(_F32).reshape(9, cin_n, src_cols)
    a = jnp.pad(a, ((0, 0), (cin_lo, cin_total - cin_lo - cin_n),
                    (dst_lo, total - dst_lo - src_cols)))
    return a.reshape(9 * cin_total, total)


def _maxpool3x3s2(z):
    H, W, C = z.shape
    zr = z.reshape(H // 2, 2, W, C)
    vh = jnp.maximum(zr[:, 0], zr[:, 1])
    prev = jnp.concatenate(
        [jnp.zeros((1, W, C), z.dtype), zr[1:, 1]], axis=0)
    vh = jnp.maximum(vh, prev)
    vr = vh.reshape(H // 2, W // 2, 2, C)
    vw = jnp.maximum(vr[:, :, 0], vr[:, :, 1])
    prevw = jnp.concatenate(
        [jnp.zeros((H // 2, 1, C), z.dtype), vr[:, 1:, 1]], axis=1)
    return jnp.maximum(vw, prevw)


def _pad_ring(v):
    H, W, C = v.shape
    zr = jnp.zeros((1, W, C), v.dtype)
    v = jnp.concatenate([zr, v, zr], axis=0)
    zc = jnp.zeros((H + 2, 1, C), v.dtype)
    return jnp.concatenate([zc, v, zc], axis=1)


def _patches3(v):
    H, W, C = v.shape
    vp = _pad_ring(v)
    taps = [vp[dy:dy + H, dx:dx + W, :] for dy in range(3) for dx in range(3)]
    return jnp.concatenate(taps, axis=-1)


def _patches3_s2(v):
    H, W, C = v.shape
    Ho, Wo = H // 2, W // 2
    vp = _pad_ring(v)
    taps = []
    for dy in range(3):
        a = vp[dy:dy + H, :, :].reshape(Ho, 2, W + 2, C)[:, 0]
        for dx in range(3):
            taps.append(a[:, dx:dx + W, :].reshape(Ho, Wo, 2, C)[:, :, 0])
    return jnp.concatenate(taps, axis=-1)


def _avgpool3x3s1(v):
    H, W, C = v.shape
    vp = _pad_ring(v)
    sh = vp[0:H, :, :] + vp[1:H + 1, :, :] + vp[2:H + 2, :, :]
    s = sh[:, 0:W, :] + sh[:, 1:W + 1, :] + sh[:, 2:W + 2, :]
    return s * (1.0 / 9.0)


def _dotb(a2d, w_ref, s_ref):
    y = jnp.dot(a2d.astype(_BF), w_ref[...], preferred_element_type=_F32)
    return y + s_ref[...]


def _softmax_pairs(y, lo, hi):
    M, C = y.shape
    lane = lax.broadcasted_iota(jnp.int32, (M, C), 1)
    even = (lane % 2) == 0
    part = jnp.where(even, pltpu.roll(y, C - 1, 1), pltpu.roll(y, 1, 1))
    sm = jax.nn.sigmoid(y - part)
    return jnp.where((lane >= lo) & (lane < hi), sm, y)


def _stem1_kernel(c_ref, w_ref, s_ref, o_ref):
    taps = [c_ref[0, dy:dy + 128, dx:dx + 128, :]
            for dy in (0, 1) for dx in (0, 1)]
    p = jnp.concatenate(taps, axis=-1)
    y = _dotb(p.reshape(16384, 256), w_ref, s_ref)
    y = jnp.maximum(y, 0.0)
    z = _maxpool3x3s2(y.reshape(128, 128, 128))
    ch = jnp.concatenate([z[..., 0:24], z[..., 64:88]], axis=-1)
    chr4 = ch.reshape(32, 2, 32, 2, 48)
    parts = [chr4[:, sy].reshape(32, 32, 2, 48)[:, :, sx]
             for sy in (0, 1) for sx in (0, 1)]
    o_ref[0] = jnp.concatenate(parts, axis=-1).astype(_BF)


def _inception_block(x3d, wf_ref, sf_ref, w2_ref, s2_ref,
                     w34_ref, s34_ref, w4b_ref, s4b_ref):
    x2 = x3d.reshape(256, 128)
    d1 = jnp.maximum(_dotb(x2, wf_ref, sf_ref), 0.0)
    pooled = _avgpool3x3s1(x3d)
    b2 = jnp.maximum(_dotb(pooled.reshape(256, 128), w2_ref, s2_ref), 0.0)
    p1 = _patches3(d1.astype(_BF).reshape(16, 16, 128))
    d2 = jnp.maximum(_dotb(p1.reshape(256, 1152), w34_ref, s34_ref), 0.0)
    p2 = _patches3(d2.astype(_BF).reshape(16, 16, 128))
    d3 = jnp.maximum(_dotb(p2.reshape(256, 1152), w4b_ref, s4b_ref), 0.0)
    lane = lax.broadcasted_iota(jnp.int32, (256, 128), 1)
    out = (jnp.where(lane < 32, d1, 0.0)
           + jnp.where((lane >= 32) & (lane < 64), b2, 0.0)
           + jnp.where((lane >= 64) & (lane < 96), d2, 0.0)
           + jnp.where(lane >= 96, d3, 0.0))
    return out.reshape(16, 16, 128)


def _tail_kernel(c_ref, w5_ref, s5_ref,
                 i1_wf, i1_sf, i1_w2, i1_s2, i1_w34, i1_s34, i1_w4b, i1_s4b,
                 i2_wf, i2_sf, i2_w2, i2_s2, i2_w34, i2_s34, i2_w4b, i2_s4b,
                 i3_wf, i3_sf, i3_w2, i3_s2, i3_w34, i3_s34, i3_w4b, i3_s4b,
                 w31_ref, s31_ref, w32_ref, s32_ref,
                 w41_ref, s41_ref, w42_ref, s42_ref,
                 h0w_ref, h0s_ref, h1w_ref, h1s_ref, h2w_ref, h2s_ref,
                 o0_ref, o1_ref, o2_ref):
    taps = [c_ref[0, dy:dy + 32, dx:dx + 32, :]
            for dy in range(3) for dx in range(3)]
    p = jnp.concatenate(taps, axis=-1)
    y = jnp.maximum(_dotb(p.reshape(1024, 1728), w5_ref, s5_ref), 0.0)
    x = _maxpool3x3s2(y.reshape(32, 32, 128))

    x = _inception_block(x, i1_wf, i1_sf, i1_w2, i1_s2,
                         i1_w34, i1_s34, i1_w4b, i1_s4b)
    x = _inception_block(x, i2_wf, i2_sf, i2_w2, i2_s2,
                         i2_w34, i2_s34, i2_w4b, i2_s4b)
    scale1 = _inception_block(x, i3_wf, i3_sf, i3_w2, i3_s2,
                              i3_w34, i3_s34, i3_w4b, i3_s4b)

    y31 = jnp.maximum(_dotb(scale1.reshape(256, 128), w31_ref, s31_ref), 0.0)
    p32 = _patches3_s2(y31.astype(_BF).reshape(16, 16, 128))
    scale2 = jnp.maximum(_dotb(p32.reshape(64, 1152), w32_ref, s32_ref), 0.0)
    y41 = jnp.maximum(_dotb(scale2.astype(_BF), w41_ref, s41_ref), 0.0)
    p42 = _patches3_s2(y41.astype(_BF).reshape(8, 8, 128))
    scale3 = jnp.maximum(_dotb(p42.reshape(16, 1152), w42_ref, s42_ref), 0.0)

    ph0 = _patches3(scale1.astype(_BF))
    h0 = _dotb(ph0.reshape(256, 1152), h0w_ref, h0s_ref)
    ph1 = _patches3(scale2.astype(_BF).reshape(8, 8, 256))
    h1 = _dotb(ph1.reshape(64, 2304), h1w_ref, h1s_ref)
    ph2 = _patches3(scale3.astype(_BF).reshape(4, 4, 256))
    h2 = _dotb(ph2.reshape(16, 2304), h2w_ref, h2s_ref)

    o0_ref[0] = _softmax_pairs(h0, 84, 126)
    o1_ref[0] = _softmax_pairs(h1, 4, 6)
    o2_ref[0] = _softmax_pairs(h2, 4, 6)


def _full_spec(shape):
    return pl.BlockSpec(shape, lambda i: (0,) * len(shape))


def _batch_spec(shape):
    return pl.BlockSpec((1,) + shape,
                        lambda i: (i,) + (0,) * len(shape))


def kernel(x,
           conv1_w, conv1_shift, conv2_w, conv2_shift,
           conv3_1_w, conv3_1_shift, conv3_2_w, conv3_2_shift,
           conv4_1_w, conv4_1_shift, conv4_2_w, conv4_2_shift,
           inc1_fused_w, inc1_fused_shift,
           inc1_branch1x1_2_w, inc1_branch1x1_2_shift,
           inc1_branch3x3_w, inc1_branch3x3_shift,
           inc1_branch3x3_2_w, inc1_branch3x3_2_shift,
           inc1_branch3x3_3_w, inc1_branch3x3_3_shift,
           inc2_fused_w, inc2_fused_shift,
           inc2_branch1x1_2_w, inc2_branch1x1_2_shift,
           inc2_branch3x3_w, inc2_branch3x3_shift,
           inc2_branch3x3_2_w, inc2_branch3x3_2_shift,
           inc2_branch3x3_3_w, inc2_branch3x3_3_shift,
           inc3_fused_w, inc3_fused_shift,
           inc3_branch1x1_2_w, inc3_branch1x1_2_shift,
           inc3_branch3x3_w, inc3_branch3x3_shift,
           inc3_branch3x3_2_w, inc3_branch3x3_2_shift,
           inc3_branch3x3_3_w, inc3_branch3x3_3_shift,
           head0_w, head0_shift, head1_w, head1_shift, head2_w, head2_shift):
    N = x.shape[0]
    cparams = pltpu.CompilerParams(dimension_semantics=("parallel",),
                                   vmem_limit_bytes=100 * 1024 * 1024)

    xc = x.astype(_BF).reshape(N, 3, 128, 4, 128, 4)
    xc = xc.transpose(0, 2, 4, 3, 5, 1).reshape(N, 128, 128, 48)
    xc = jnp.pad(xc, ((0, 0), (1, 0), (1, 0), (0, 16)))

    w4 = _gather_rows(conv1_w.astype(_F32), _CONV1_IDX)
    w4c, s4c = _crelu_pack(w4, conv1_shift)

    cells2 = pl.pallas_call(
        _stem1_kernel,
        out_shape=jax.ShapeDtypeStruct((N, 32, 32, 192), _BF),
        grid=(N,),
        in_specs=[_batch_spec((129, 129, 64)),
                  _full_spec((256, 128)), _full_spec((1, 128))],
        out_specs=_batch_spec((32, 32, 192)),
        compiler_params=cparams,
    )(xc, w4c, s4c)
    cells2 = jnp.pad(cells2, ((0, 0), (1, 1), (1, 1), (0, 0)))

    w5 = _gather_rows(conv2_w.astype(_F32), _CONV2_IDX)
    w5c, s5c = _crelu_pack(w5, conv2_shift)

    def inc_prep(fw, fs, b2w, b2s, b3w, b3s, b32w, b32s, b33w, b33s):
        w2m, s2m = _band(b2w, b2s, 32, 32)
        w34 = (_embed_rows(b3w, 32, 32, 24, 128, 64)
               + _embed_rows(b32w, 32, 56, 24, 128, 96))
        s34 = (jnp.pad(b3s[:, :32], ((0, 0), (64, 32)))
               + jnp.pad(b32s[:, :32], ((0, 0), (96, 0))))
        w4b = _embed_rows(b33w, 32, 96, 32, 128, 96)
        s4b = jnp.pad(b33s[:, :32], ((0, 0), (96, 0)))
        return (fw.astype(_BF), fs, w2m, s2m, w34.astype(_BF), s34,
                w4b.astype(_BF), s4b)

    inc1 = inc_prep(inc1_fused_w, inc1_fused_shift,
                    inc1_branch1x1_2_w, inc1_branch1x1_2_shift,
                    inc1_branch3x3_w, inc1_branch3x3_shift,
                    inc1_branch3x3_2_w, inc1_branch3x3_2_shift,
                    inc1_branch3x3_3_w, inc1_branch3x3_3_shift)
    inc2 = inc_prep(inc2_fused_w, inc2_fused_shift,
                    inc2_branch1x1_2_w, inc2_branch1x1_2_shift,
                    inc2_branch3x3_w, inc2_branch3x3_shift,
                    inc2_branch3x3_2_w, inc2_branch3x3_2_shift,
                    inc2_branch3x3_3_w, inc2_branch3x3_3_shift)
    inc3 = inc_prep(inc3_fused_w, inc3_fused_shift,
                    inc3_branch1x1_2_w, inc3_branch1x1_2_shift,
                    inc3_branch3x3_w, inc3_branch3x3_shift,
                    inc3_branch3x3_2_w, inc3_branch3x3_2_shift,
                    inc3_branch3x3_3_w, inc3_branch3x3_3_shift)

    ins = ([cells2, w5c, s5c] + list(inc1) + list(inc2) + list(inc3)
           + [conv3_1_w, conv3_1_shift, conv3_2_w, conv3_2_shift,
              conv4_1_w, conv4_1_shift, conv4_2_w, conv4_2_shift,
              head0_w, head0_shift, head1_w, head1_shift,
              head2_w, head2_shift])
    in_specs = [_batch_spec((34, 34, 192))]
    for a in ins[1:]:
        in_specs.append(_full_spec(a.shape))

    o0, o1, o2 = pl.pallas_call(
        _tail_kernel,
        out_shape=[jax.ShapeDtypeStruct((N, 256, 128), _F32),
                   jax.ShapeDtypeStruct((N, 64, 128), _F32),
                   jax.ShapeDtypeStruct((N, 16, 128), _F32)],
        grid=(N,),
        in_specs=in_specs,
        out_specs=[_batch_spec((256, 128)),
                   _batch_spec((64, 128)),
                   _batch_spec((16, 128))],
        compiler_params=cparams,
    )(*ins)

    loc = jnp.concatenate([o0[:, :, :84].reshape(N, -1),
                           o1[:, :, :4].reshape(N, -1),
                           o2[:, :, :4].reshape(N, -1)], axis=1)
    conf = jnp.concatenate([o0[:, :, 84:126].reshape(N, -1),
                            o1[:, :, 4:6].reshape(N, -1),
                            o2[:, :, 4:6].reshape(N, -1)], axis=1)
    return loc.reshape(N, -1, 4), conf.reshape(N, -1, 2)


# speedup vs baseline: 43.8503x; 1.0901x over previous
import functools

import numpy as np

import jax
import jax.numpy as jnp
from jax import lax
from jax.experimental import pallas as pl
from jax.experimental.pallas import tpu as pltpu

_BF = jnp.bfloat16
_F32 = jnp.float32


def _remap_rows(w, idx):
    P = np.zeros((len(idx), w.shape[0]), np.float32)
    for i, j in enumerate(idx):
        if j >= 0:
            P[i, j] = 1.0
    return jnp.dot(jnp.asarray(P), w)


def _conv1_taps():
    idx = np.full((192,), -1, np.int32)
    for ky in range(7):
        for kx in range(7):
            dy, sy = (ky + 1) // 4, (ky + 1) % 4
            dx, sx = (kx + 1) // 4, (kx + 1) % 4
            for c in range(3):
                idx[(dy * 2 + dx) * 48 + c * 16 + sy * 4 + sx] = (ky * 7 + kx) * 3 + c
    return idx


_CONV1_IDX = _conv1_taps()


def _conv2_taps():
    idx = np.full((1728,), -1, np.int32)
    for ky in range(5):
        for kx in range(5):
            dy, sy = ky // 2, ky % 2
            dx, sx = kx // 2, kx % 2
            for c in range(48):
                idx[(dy * 3 + dx) * 192 + sy * 96 + sx * 48 + c] = (ky * 5 + kx) * 48 + c
    return idx


_CONV2_IDX = _conv2_taps()


def _crelu_pack(w, shift):
    w2 = jnp.concatenate([w, -w], axis=1).astype(_BF)
    s2 = jnp.concatenate([shift, -shift], axis=1)
    return w2, s2


def _band(w, s, src_cols, dst_lo, total=128):
    wb = jnp.pad(w[:, :src_cols].astype(_F32),
                 ((0, 0), (dst_lo, total - dst_lo - src_cols)))
    sb = jnp.pad(s[:, :src_cols], ((0, 0), (dst_lo, total - dst_lo - src_cols)))
    return wb.astype(_BF), sb


def _embed_rows(w, src_cols, cin_lo, cin_n, cin_total, dst_lo, total=128):
    a = w[:, :src_cols].astype(_F32).reshape(9, cin_n, src_cols)
    a = jnp.pad(a, ((0, 0), (cin_lo, cin_total - cin_lo - cin_n),
                    (dst_lo, total - dst_lo - src_cols)))
    return a.reshape(9 * cin_total, total)


def _maxpool3x3s2(z):
    H, W, C = z.shape
    zr = z.reshape(H // 2, 2, W, C)
    vh = jnp.maximum(zr[:, 0], zr[:, 1])
    prev = jnp.concatenate(
        [jnp.zeros((1, W, C), z.dtype), zr[1:, 1]], axis=0)
    vh = jnp.maximum(vh, prev)
    vr = vh.reshape(H // 2, W // 2, 2, C)
    vw = jnp.maximum(vr[:, :, 0], vr[:, :, 1])
    prevw = jnp.concatenate(
        [jnp.zeros((H // 2, 1, C), z.dtype), vr[:, 1:, 1]], axis=1)
    return jnp.maximum(vw, prevw)


def _pad_ring(v):
    H, W, C = v.shape
    zr = jnp.zeros((1, W, C), v.dtype)
    v = jnp.concatenate([zr, v, zr], axis=0)
    zc = jnp.zeros((H + 2, 1, C), v.dtype)
    return jnp.concatenate([zc, v, zc], axis=1)


def _patches3(v):
    H, W, C = v.shape
    vp = _pad_ring(v)
    taps = [vp[dy:dy + H, dx:dx + W, :] for dy in range(3) for dx in range(3)]
    return jnp.concatenate(taps, axis=-1)


def _patches3_s2(v):
    H, W, C = v.shape
    Ho, Wo = H // 2, W // 2
    vp = _pad_ring(v)
    taps = []
    for dy in range(3):
        a = vp[dy:dy + H, :, :].reshape(Ho, 2, W + 2, C)[:, 0]
        for dx in range(3):
            taps.append(a[:, dx:dx + W, :].reshape(Ho, Wo, 2, C)[:, :, 0])
    return jnp.concatenate(taps, axis=-1)


def _avgpool3x3s1(v):
    H, W, C = v.shape
    vp = _pad_ring(v)
    sh = vp[0:H, :, :] + vp[1:H + 1, :, :] + vp[2:H + 2, :, :]
    s = sh[:, 0:W, :] + sh[:, 1:W + 1, :] + sh[:, 2:W + 2, :]
    return s * (1.0 / 9.0)


def _dotb(a2d, w_ref, s_ref):
    y = jnp.dot(a2d.astype(_BF), w_ref[...], preferred_element_type=_F32)
    return y + s_ref[...]


def _softmax_pairs(y, lo, hi):
    M, C = y.shape
    lane = lax.broadcasted_iota(jnp.int32, (M, C), 1)
    even = (lane % 2) == 0
    part = jnp.where(even, pltpu.roll(y, C - 1, 1), pltpu.roll(y, 1, 1))
    sm = jax.nn.sigmoid(y - part)
    return jnp.where((lane >= lo) & (lane < hi), sm, y)


def _shift_down(v, axis):
    H, W, C = v.shape
    if axis == 0:
        return jnp.concatenate([jnp.zeros((1, W, C), v.dtype), v[:-1]], axis=0)
    return jnp.concatenate([jnp.zeros((H, 1, C), v.dtype), v[:, :-1]], axis=1)


def _shift_up(v, axis):
    H, W, C = v.shape
    if axis == 0:
        return jnp.concatenate([v[1:], jnp.zeros((1, W, C), v.dtype)], axis=0)
    return jnp.concatenate([v[:, 1:], jnp.zeros((H, 1, C), v.dtype)], axis=1)


def _stem1_kernel(c_ref, w_ref, s_ref, o_ref):
    v = c_ref[0]
    rows = [_shift_down(v, 0), v]
    taps = []
    for r in rows:
        taps.append(_shift_down(r, 1))
        taps.append(r)
    p = jnp.concatenate(taps, axis=-1)
    y = _dotb(p.reshape(16384, 192), w_ref, s_ref)
    y = jnp.maximum(y, 0.0)
    z = _maxpool3x3s2(y.reshape(128, 128, 128))
    ch = jnp.concatenate([z[..., 0:24], z[..., 64:88]], axis=-1)
    chr4 = ch.reshape(32, 2, 32, 2, 48)
    parts = [chr4[:, sy].reshape(32, 32, 2, 48)[:, :, sx]
             for sy in (0, 1) for sx in (0, 1)]
    o_ref[0] = jnp.concatenate(parts, axis=-1).astype(_BF)


def _inception_block(x3d, wf_ref, sf_ref, w2_ref, s2_ref,
                     w34_ref, s34_ref, w4b_ref, s4b_ref):
    x2 = x3d.reshape(256, 128)
    d1 = jnp.maximum(_dotb(x2, wf_ref, sf_ref), 0.0)
    pooled = _avgpool3x3s1(x3d)
    b2 = jnp.maximum(_dotb(pooled.reshape(256, 128), w2_ref, s2_ref), 0.0)
    p1 = _patches3(d1.astype(_BF).reshape(16, 16, 128))
    d2 = jnp.maximum(_dotb(p1.reshape(256, 1152), w34_ref, s34_ref), 0.0)
    p2 = _patches3(d2.astype(_BF).reshape(16, 16, 128))
    d3 = jnp.maximum(_dotb(p2.reshape(256, 1152), w4b_ref, s4b_ref), 0.0)
    lane = lax.broadcasted_iota(jnp.int32, (256, 128), 1)
    out = (jnp.where(lane < 32, d1, 0.0)
           + jnp.where((lane >= 32) & (lane < 64), b2, 0.0)
           + jnp.where((lane >= 64) & (lane < 96), d2, 0.0)
           + jnp.where(lane >= 96, d3, 0.0))
    return out.reshape(16, 16, 128)


def _tail_kernel(c_ref, w5_ref, s5_ref,
                 i1_wf, i1_sf, i1_w2, i1_s2, i1_w34, i1_s34, i1_w4b, i1_s4b,
                 i2_wf, i2_sf, i2_w2, i2_s2, i2_w34, i2_s34, i2_w4b, i2_s4b,
                 i3_wf, i3_sf, i3_w2, i3_s2, i3_w34, i3_s34, i3_w4b, i3_s4b,
                 w31_ref, s31_ref, w32_ref, s32_ref,
                 w41_ref, s41_ref, w42_ref, s42_ref,
                 h0w_ref, h0s_ref, h1w_ref, h1s_ref, h2w_ref, h2s_ref,
                 o0_ref, o1_ref, o2_ref):
    v = c_ref[0]
    rows = [_shift_down(v, 0), v, _shift_up(v, 0)]
    taps = []
    for r in rows:
        taps.extend([_shift_down(r, 1), r, _shift_up(r, 1)])
    p = jnp.concatenate(taps, axis=-1)
    y = jnp.maximum(_dotb(p.reshape(1024, 1728), w5_ref, s5_ref), 0.0)
    x = _maxpool3x3s2(y.reshape(32, 32, 128))

    x = _inception_block(x, i1_wf, i1_sf, i1_w2, i1_s2,
                         i1_w34, i1_s34, i1_w4b, i1_s4b)
    x = _inception_block(x, i2_wf, i2_sf, i2_w2, i2_s2,
                         i2_w34, i2_s34, i2_w4b, i2_s4b)
    scale1 = _inception_block(x, i3_wf, i3_sf, i3_w2, i3_s2,
                              i3_w34, i3_s34, i3_w4b, i3_s4b)

    y31 = jnp.maximum(_dotb(scale1.reshape(256, 128), w31_ref, s31_ref), 0.0)
    p32 = _patches3_s2(y31.astype(_BF).reshape(16, 16, 128))
    scale2 = jnp.maximum(_dotb(p32.reshape(64, 1152), w32_ref, s32_ref), 0.0)
    y41 = jnp.maximum(_dotb(scale2.astype(_BF), w41_ref, s41_ref), 0.0)
    p42 = _patches3_s2(y41.astype(_BF).reshape(8, 8, 128))
    scale3 = jnp.maximum(_dotb(p42.reshape(16, 1152), w42_ref, s42_ref), 0.0)

    ph0 = _patches3(scale1.astype(_BF))
    h0 = _dotb(ph0.reshape(256, 1152), h0w_ref, h0s_ref)
    ph1 = _patches3(scale2.astype(_BF).reshape(8, 8, 256))
    h1 = _dotb(ph1.reshape(64, 2304), h1w_ref, h1s_ref)
    ph2 = _patches3(scale3.astype(_BF).reshape(4, 4, 256))
    h2 = _dotb(ph2.reshape(16, 2304), h2w_ref, h2s_ref)

    o0_ref[0] = _softmax_pairs(h0, 84, 126)
    o1_ref[0] = _softmax_pairs(h1, 4, 6)
    o2_ref[0] = _softmax_pairs(h2, 4, 6)


def _full_spec(shape):
    return pl.BlockSpec(shape, lambda i: (0,) * len(shape))


def _batch_spec(shape):
    return pl.BlockSpec((1,) + shape,
                        lambda i: (i,) + (0,) * len(shape))


def kernel(x,
           conv1_w, conv1_shift, conv2_w, conv2_shift,
           conv3_1_w, conv3_1_shift, conv3_2_w, conv3_2_shift,
           conv4_1_w, conv4_1_shift, conv4_2_w, conv4_2_shift,
           inc1_fused_w, inc1_fused_shift,
           inc1_branch1x1_2_w, inc1_branch1x1_2_shift,
           inc1_branch3x3_w, inc1_branch3x3_shift,
           inc1_branch3x3_2_w, inc1_branch3x3_2_shift,
           inc1_branch3x3_3_w, inc1_branch3x3_3_shift,
           inc2_fused_w, inc2_fused_shift,
           inc2_branch1x1_2_w, inc2_branch1x1_2_shift,
           inc2_branch3x3_w, inc2_branch3x3_shift,
           inc2_branch3x3_2_w, inc2_branch3x3_2_shift,
           inc2_branch3x3_3_w, inc2_branch3x3_3_shift,
           inc3_fused_w, inc3_fused_shift,
           inc3_branch1x1_2_w, inc3_branch1x1_2_shift,
           inc3_branch3x3_w, inc3_branch3x3_shift,
           inc3_branch3x3_2_w, inc3_branch3x3_2_shift,
           inc3_branch3x3_3_w, inc3_branch3x3_3_shift,
           head0_w, head0_shift, head1_w, head1_shift, head2_w, head2_shift):
    N = x.shape[0]
    cparams = pltpu.CompilerParams(dimension_semantics=("parallel",),
                                   vmem_limit_bytes=100 * 1024 * 1024)

    xc = x.astype(_BF).reshape(N, 3, 128, 4, 128, 4)
    xc = xc.transpose(0, 2, 4, 1, 3, 5).reshape(N, 128, 128, 48)

    w4 = _remap_rows(conv1_w.astype(_F32), _CONV1_IDX)
    w4c, s4c = _crelu_pack(w4, conv1_shift)

    cells2 = pl.pallas_call(
        _stem1_kernel,
        out_shape=jax.ShapeDtypeStruct((N, 32, 32, 192), _BF),
        grid=(N,),
        in_specs=[_batch_spec((128, 128, 48)),
                  _full_spec((192, 128)), _full_spec((1, 128))],
        out_specs=_batch_spec((32, 32, 192)),
        compiler_params=cparams,
    )(xc, w4c, s4c)

    w5 = _remap_rows(conv2_w.astype(_F32), _CONV2_IDX)
    w5c, s5c = _crelu_pack(w5, conv2_shift)

    def inc_prep(fw, fs, b2w, b2s, b3w, b3s, b32w, b32s, b33w, b33s):
        w2m, s2m = _band(b2w, b2s, 32, 32)
        w34 = (_embed_rows(b3w, 32, 32, 24, 128, 64)
               + _embed_rows(b32w, 32, 56, 24, 128, 96))
        s34 = (jnp.pad(b3s[:, :32], ((0, 0), (64, 32)))
               + jnp.pad(b32s[:, :32], ((0, 0), (96, 0))))
        w4b = _embed_rows(b33w, 32, 96, 32, 128, 96)
        s4b = jnp.pad(b33s[:, :32], ((0, 0), (96, 0)))
        return (fw.astype(_BF), fs, w2m, s2m, w34.astype(_BF), s34,
                w4b.astype(_BF), s4b)

    inc1 = inc_prep(inc1_fused_w, inc1_fused_shift,
                    inc1_branch1x1_2_w, inc1_branch1x1_2_shift,
                    inc1_branch3x3_w, inc1_branch3x3_shift,
                    inc1_branch3x3_2_w, inc1_branch3x3_2_shift,
                    inc1_branch3x3_3_w, inc1_branch3x3_3_shift)
    inc2 = inc_prep(inc2_fused_w, inc2_fused_shift,
                    inc2_branch1x1_2_w, inc2_branch1x1_2_shift,
                    inc2_branch3x3_w, inc2_branch3x3_shift,
                    inc2_branch3x3_2_w, inc2_branch3x3_2_shift,
                    inc2_branch3x3_3_w, inc2_branch3x3_3_shift)
    inc3 = inc_prep(inc3_fused_w, inc3_fused_shift,
                    inc3_branch1x1_2_w, inc3_branch1x1_2_shift,
                    inc3_branch3x3_w, inc3_branch3x3_shift,
                    inc3_branch3x3_2_w, inc3_branch3x3_2_shift,
                    inc3_branch3x3_3_w, inc3_branch3x3_3_shift)

    ins = ([cells2, w5c, s5c] + list(inc1) + list(inc2) + list(inc3)
           + [conv3_1_w, conv3_1_shift, conv3_2_w, conv3_2_shift,
              conv4_1_w, conv4_1_shift, conv4_2_w, conv4_2_shift,
              head0_w, head0_shift, head1_w, head1_shift,
              head2_w, head2_shift])
    in_specs = [_batch_spec((32, 32, 192))]
    for a in ins[1:]:
        in_specs.append(_full_spec(a.shape))

    o0, o1, o2 = pl.pallas_call(
        _tail_kernel,
        out_shape=[jax.ShapeDtypeStruct((N, 256, 128), _F32),
                   jax.ShapeDtypeStruct((N, 64, 128), _F32),
                   jax.ShapeDtypeStruct((N, 16, 128), _F32)],
        grid=(N,),
        in_specs=in_specs,
        out_specs=[_batch_spec((256, 128)),
                   _batch_spec((64, 128)),
                   _batch_spec((16, 128))],
        compiler_params=cparams,
    )(*ins)

    loc = jnp.concatenate([o0[:, :, :84].reshape(N, -1),
                           o1[:, :, :4].reshape(N, -1),
                           o2[:, :, :4].reshape(N, -1)], axis=1)
    conf = jnp.concatenate([o0[:, :, 84:126].reshape(N, -1),
                            o1[:, :, 4:6].reshape(N, -1),
                            o2[:, :, 4:6].reshape(N, -1)], axis=1)
    return loc.reshape(N, -1, 4), conf.reshape(N, -1, 2)
